# Initial kernel scaffold; baseline (speedup 1.0000x reference)
import jax
import jax.numpy as jnp
from jax.experimental import pallas as pl


def kernel(inputs_0, inputs_1, emb0, emb1):
    raise NotImplementedError("write your pallas kernel here")


# speedup vs baseline: 1.0694x; 1.0694x over previous
import functools

import jax
import jax.numpy as jnp
from jax import lax
from jax.experimental import pallas as pl
from jax.experimental.pallas import tpu as pltpu
from jax.experimental.pallas import tpu_sc as plsc

_COMMITMENT_COST = 0.25
_CHUNK = 2048


def _argmin_body(xb_ref, ebT_ref, x2_ref, e2_ref, idx_ref, loss_ref, *,
                 num_codes, bf16_acc):
    i = pl.program_id(0)
    xb = xb_ref[...]
    ebT = ebT_ref[...]
    mm = jnp.dot(xb, ebT, preferred_element_type=jnp.float32)
    dist = (x2_ref[...] + e2_ref[...]) - 2.0 * mm

    if not bf16_acc:
        acc_val = jnp.min(dist, axis=1)
        iota = lax.broadcasted_iota(jnp.int32, dist.shape, 1)
        acc_idx = jnp.min(
            jnp.where(dist == acc_val[:, None], iota, num_codes), axis=1)
    else:
        nchunks = num_codes // _CHUNK
        acc = None
        for c in range(nchunks):
            dc = dist[:, c * _CHUNK:(c + 1) * _CHUNK]
            vc = jnp.min(dc, axis=1)
            iota = lax.broadcasted_iota(jnp.int32, dc.shape, 1) + c * _CHUNK
            ic = jnp.min(jnp.where(dc == vc[:, None], iota, num_codes), axis=1)
            if acc is None:
                acc_idx, acc_val, acc_bf = ic, vc, vc.astype(jnp.bfloat16)
                acc = True
            else:
                upd = vc < acc_bf.astype(jnp.float32)
                acc_idx = jnp.where(upd, ic, acc_idx)
                acc_val = jnp.where(upd, vc, acc_val)
                acc_bf = jnp.where(upd, vc.astype(jnp.bfloat16), acc_bf)
    idx_ref[0, 0, :] = acc_idx

    @pl.when(i == 0)
    def _():
        loss_ref[0, 0] = 0.0

    loss_ref[0, 0] += jnp.sum(acc_val)


def _argmin_level(xb, ebT, x2, e2, nt, bf16_acc):
    n, d = xb.shape
    k = ebT.shape[1]
    nb = n // nt
    idx3, loss = pl.pallas_call(
        functools.partial(_argmin_body, num_codes=k, bf16_acc=bf16_acc),
        grid=(nb,),
        in_specs=[
            pl.BlockSpec((nt, d), lambda i: (i, 0)),
            pl.BlockSpec((d, k), lambda i: (0, 0)),
            pl.BlockSpec((nt, 1), lambda i: (i, 0)),
            pl.BlockSpec((1, k), lambda i: (0, 0)),
        ],
        out_specs=[
            pl.BlockSpec((1, 1, nt), lambda i: (i, 0, 0)),
            pl.BlockSpec((1, 1), lambda i: (0, 0), memory_space=pltpu.SMEM),
        ],
        out_shape=[
            jax.ShapeDtypeStruct((nb, 1, nt), jnp.int32),
            jax.ShapeDtypeStruct((1, 1), jnp.float32),
        ],
    )(xb, ebT, x2, e2)
    return idx3.reshape(n), loss[0, 0]


def _sc_gather(emb, idx):
    n = idx.shape[0]
    d = emb.shape[1]
    info = plsc.get_sparse_core_info()
    nc = info.num_cores
    nw = nc * info.num_subcores
    b_per_w = n // nw
    chunk = min(128, b_per_w)
    n_chunks = b_per_w // chunk
    mesh = plsc.VectorSubcoreMesh(core_axis_name="c", subcore_axis_name="s")

    @functools.partial(
        pl.kernel,
        mesh=mesh,
        out_type=jax.ShapeDtypeStruct((n, d), jnp.float32),
        compiler_params=pltpu.CompilerParams(use_tc_tiling_on_sc=False),
        scratch_types=[
            pltpu.VMEM((n_chunks, chunk), jnp.int32),
            pltpu.VMEM((b_per_w, d), jnp.float32),
            pltpu.SemaphoreType.DMA,
        ],
    )
    def gather_k(table_hbm, idx_hbm, out_hbm, idx_v, rows_v, sem):
        wid = lax.axis_index("s") * nc + lax.axis_index("c")
        base = wid * b_per_w
        for c in range(n_chunks):
            pltpu.sync_copy(idx_hbm.at[pl.ds(base + c * chunk, chunk)],
                            idx_v.at[c])
            pltpu.async_copy(table_hbm.at[idx_v.at[c]],
                             rows_v.at[pl.ds(c * chunk, chunk)], sem).wait()
        pltpu.sync_copy(rows_v, out_hbm.at[pl.ds(base, b_per_w)])

    return gather_k(emb, idx)


def _vq_level_setup(x, emb):
    c = x.shape[1]
    flat = jnp.transpose(x, (0, 2, 3, 1)).reshape(-1, c)
    x2 = jnp.sum(flat ** 2, axis=1, keepdims=True)
    e2 = jnp.sum(emb ** 2, axis=1)[None, :]
    return flat.astype(jnp.bfloat16), emb.T.astype(jnp.bfloat16), x2, e2


def kernel(inputs_0, inputs_1, emb0, emb1):
    b0, c0, h0, w0 = inputs_0.shape
    b1, c1, h1, w1 = inputs_1.shape

    idx0, lsum0 = _argmin_level(*_vq_level_setup(inputs_0, emb0), 256, True)
    idx1, lsum1 = _argmin_level(*_vq_level_setup(inputs_1, emb1), 256, False)

    q0 = _sc_gather(emb0, idx0).reshape(b0, h0, w0, c0).transpose(0, 3, 1, 2)
    q1 = _sc_gather(emb1, idx1).reshape(b1, h1, w1, c1).transpose(0, 3, 1, 2)

    n0 = b0 * h0 * w0
    n1 = b1 * h1 * w1
    total_loss = _COMMITMENT_COST * (lsum0 / (n0 * c0) + lsum1 / (n1 * c1))
    return total_loss, q0, q1


# speedup vs baseline: 1.1188x; 1.0462x over previous
import functools

import jax
import jax.numpy as jnp
from jax import lax
from jax.experimental import pallas as pl
from jax.experimental.pallas import tpu as pltpu
from jax.experimental.pallas import tpu_sc as plsc

_COMMITMENT_COST = 0.25
_CHUNK = 2048


def _argmin_body(xb_ref, ebT_ref, x2_ref, e2_ref, idx_ref, loss_ref, *,
                 num_codes, bf16_acc):
    i = pl.program_id(0)
    xb = xb_ref[...]
    ebT = ebT_ref[...]
    mm = jnp.dot(xb, ebT, preferred_element_type=jnp.float32)
    dist = (x2_ref[...] + e2_ref[...]) - 2.0 * mm

    if not bf16_acc:
        acc_val = jnp.min(dist, axis=1)
        iota = lax.broadcasted_iota(jnp.int32, dist.shape, 1)
        acc_idx = jnp.min(
            jnp.where(dist == acc_val[:, None], iota, num_codes), axis=1)
    else:
        nchunks = num_codes // _CHUNK
        acc = None
        for c in range(nchunks):
            dc = dist[:, c * _CHUNK:(c + 1) * _CHUNK]
            vc = jnp.min(dc, axis=1)
            iota = lax.broadcasted_iota(jnp.int32, dc.shape, 1) + c * _CHUNK
            ic = jnp.min(jnp.where(dc == vc[:, None], iota, num_codes), axis=1)
            if acc is None:
                acc_idx, acc_val, acc_bf = ic, vc, vc.astype(jnp.bfloat16)
                acc = True
            else:
                upd = vc < acc_bf.astype(jnp.float32)
                acc_idx = jnp.where(upd, ic, acc_idx)
                acc_val = jnp.where(upd, vc, acc_val)
                acc_bf = jnp.where(upd, vc.astype(jnp.bfloat16), acc_bf)
    idx_ref[0, 0, :] = acc_idx

    @pl.when(i == 0)
    def _():
        loss_ref[0, 0] = 0.0

    loss_ref[0, 0] += jnp.sum(acc_val)


def _argmin_level(xb, ebT, x2, e2, nt, bf16_acc):
    n, d = xb.shape
    k = ebT.shape[1]
    nb = n // nt
    idx3, loss = pl.pallas_call(
        functools.partial(_argmin_body, num_codes=k, bf16_acc=bf16_acc),
        grid=(nb,),
        in_specs=[
            pl.BlockSpec((nt, d), lambda i: (i, 0)),
            pl.BlockSpec((d, k), lambda i: (0, 0)),
            pl.BlockSpec((nt, 1), lambda i: (i, 0)),
            pl.BlockSpec((1, k), lambda i: (0, 0)),
        ],
        out_specs=[
            pl.BlockSpec((1, 1, nt), lambda i: (i, 0, 0)),
            pl.BlockSpec((1, 1), lambda i: (0, 0), memory_space=pltpu.SMEM),
        ],
        out_shape=[
            jax.ShapeDtypeStruct((nb, 1, nt), jnp.int32),
            jax.ShapeDtypeStruct((1, 1), jnp.float32),
        ],
    )(xb, ebT, x2, e2)
    return idx3.reshape(n), loss[0, 0]


def _sc_gather(emb, idx):
    n = idx.shape[0]
    d = emb.shape[1]
    info = plsc.get_sparse_core_info()
    nc = info.num_cores
    nw = nc * info.num_subcores
    b_per_w = n // nw
    chunk = min(128, b_per_w)
    n_chunks = b_per_w // chunk
    mesh = plsc.VectorSubcoreMesh(core_axis_name="c", subcore_axis_name="s")

    @functools.partial(
        pl.kernel,
        mesh=mesh,
        out_type=jax.ShapeDtypeStruct((n, d), jnp.float32),
        compiler_params=pltpu.CompilerParams(use_tc_tiling_on_sc=False),
        scratch_types=[
            pltpu.VMEM((n_chunks, chunk), jnp.int32),
            pltpu.VMEM((b_per_w, d), jnp.float32),
            pltpu.SemaphoreType.DMA,
        ],
    )
    def gather_k(table_hbm, idx_hbm, out_hbm, idx_v, rows_v, sem):
        wid = lax.axis_index("s") * nc + lax.axis_index("c")
        base = wid * b_per_w
        for c in range(n_chunks):
            pltpu.sync_copy(idx_hbm.at[pl.ds(base + c * chunk, chunk)],
                            idx_v.at[c])
            pltpu.async_copy(table_hbm.at[idx_v.at[c]],
                             rows_v.at[pl.ds(c * chunk, chunk)], sem).wait()
        pltpu.sync_copy(rows_v, out_hbm.at[pl.ds(base, b_per_w)])

    return gather_k(emb, idx)


def _vq_level_setup(x, emb):
    c = x.shape[1]
    flat = jnp.transpose(x, (0, 2, 3, 1)).reshape(-1, c)
    x2 = jnp.sum(flat ** 2, axis=1, keepdims=True)
    e2 = jnp.sum(emb ** 2, axis=1)[None, :]
    return flat.astype(jnp.bfloat16), emb.T.astype(jnp.bfloat16), x2, e2


def kernel(inputs_0, inputs_1, emb0, emb1):
    b0, c0, h0, w0 = inputs_0.shape
    b1, c1, h1, w1 = inputs_1.shape

    idx0, lsum0 = _argmin_level(*_vq_level_setup(inputs_0, emb0), 512, True)
    idx1, lsum1 = _argmin_level(*_vq_level_setup(inputs_1, emb1), 512, False)

    q0 = _sc_gather(emb0, idx0).reshape(b0, h0, w0, c0).transpose(0, 3, 1, 2)
    q1 = _sc_gather(emb1, idx1).reshape(b1, h1, w1, c1).transpose(0, 3, 1, 2)

    n0 = b0 * h0 * w0
    n1 = b1 * h1 * w1
    total_loss = _COMMITMENT_COST * (lsum0 / (n0 * c0) + lsum1 / (n1 * c1))
    return total_loss, q0, q1


# speedup vs baseline: 1.1261x; 1.0065x over previous
import functools

import jax
import jax.numpy as jnp
from jax import lax
from jax.experimental import pallas as pl
from jax.experimental.pallas import tpu as pltpu
from jax.experimental.pallas import tpu_sc as plsc

_COMMITMENT_COST = 0.25
_CHUNK = 2048


def _argmin_body(xb_ref, ebT_ref, x2_ref, e2_ref, idx_ref, loss_ref, *,
                 num_codes, bf16_acc):
    i = pl.program_id(0)
    xb = xb_ref[...]
    ebT = ebT_ref[...]
    mm = jnp.dot(xb, ebT, preferred_element_type=jnp.float32)
    dist = (x2_ref[...] + e2_ref[...]) - 2.0 * mm

    if not bf16_acc:
        acc_val = jnp.min(dist, axis=1)
        iota = lax.broadcasted_iota(jnp.int32, dist.shape, 1)
        acc_idx = jnp.min(
            jnp.where(dist == acc_val[:, None], iota, num_codes), axis=1)
    else:
        nchunks = num_codes // _CHUNK
        acc = None
        for c in range(nchunks):
            dc = dist[:, c * _CHUNK:(c + 1) * _CHUNK]
            vc = jnp.min(dc, axis=1)
            iota = lax.broadcasted_iota(jnp.int32, dc.shape, 1) + c * _CHUNK
            ic = jnp.min(jnp.where(dc == vc[:, None], iota, num_codes), axis=1)
            if acc is None:
                acc_idx, acc_val, acc_bf = ic, vc, vc.astype(jnp.bfloat16)
                acc = True
            else:
                upd = vc < acc_bf.astype(jnp.float32)
                acc_idx = jnp.where(upd, ic, acc_idx)
                acc_val = jnp.where(upd, vc, acc_val)
                acc_bf = jnp.where(upd, vc.astype(jnp.bfloat16), acc_bf)
    idx_ref[0, 0, :] = acc_idx

    @pl.when(i == 0)
    def _():
        loss_ref[0, 0] = 0.0

    loss_ref[0, 0] += jnp.sum(acc_val)


def _argmin_level(xb, ebT, x2, e2, nt, bf16_acc):
    n, d = xb.shape
    k = ebT.shape[1]
    nb = n // nt
    idx3, loss = pl.pallas_call(
        functools.partial(_argmin_body, num_codes=k, bf16_acc=bf16_acc),
        grid=(nb,),
        in_specs=[
            pl.BlockSpec((nt, d), lambda i: (i, 0)),
            pl.BlockSpec((d, k), lambda i: (0, 0)),
            pl.BlockSpec((nt, 1), lambda i: (i, 0)),
            pl.BlockSpec((1, k), lambda i: (0, 0)),
        ],
        out_specs=[
            pl.BlockSpec((1, 1, nt), lambda i: (i, 0, 0)),
            pl.BlockSpec((1, 1), lambda i: (0, 0), memory_space=pltpu.SMEM),
        ],
        out_shape=[
            jax.ShapeDtypeStruct((nb, 1, nt), jnp.int32),
            jax.ShapeDtypeStruct((1, 1), jnp.float32),
        ],
    )(xb, ebT, x2, e2)
    return idx3.reshape(n), loss[0, 0]


def _sc_gather(emb, idx):
    n = idx.shape[0]
    d = emb.shape[1]
    info = plsc.get_sparse_core_info()
    nc = info.num_cores
    nw = nc * info.num_subcores
    b_per_w = n // nw
    chunk = min(128, b_per_w)
    n_chunks = b_per_w // chunk
    mesh = plsc.VectorSubcoreMesh(core_axis_name="c", subcore_axis_name="s")

    @functools.partial(
        pl.kernel,
        mesh=mesh,
        out_type=jax.ShapeDtypeStruct((n, d), jnp.float32),
        compiler_params=pltpu.CompilerParams(use_tc_tiling_on_sc=False),
        scratch_types=[
            pltpu.VMEM((n_chunks, chunk), jnp.int32),
            pltpu.VMEM((b_per_w, d), jnp.float32),
            pltpu.SemaphoreType.DMA,
        ],
    )
    def gather_k(table_hbm, idx_hbm, out_hbm, idx_v, rows_v, sem):
        wid = lax.axis_index("s") * nc + lax.axis_index("c")
        base = wid * b_per_w
        for c in range(n_chunks):
            pltpu.sync_copy(idx_hbm.at[pl.ds(base + c * chunk, chunk)],
                            idx_v.at[c])
            pltpu.async_copy(table_hbm.at[idx_v.at[c]],
                             rows_v.at[pl.ds(c * chunk, chunk)], sem).wait()
        pltpu.sync_copy(rows_v, out_hbm.at[pl.ds(base, b_per_w)])

    return gather_k(emb, idx)


def _vq_level_setup(x, emb):
    c = x.shape[1]
    flat = jnp.transpose(x, (0, 2, 3, 1)).reshape(-1, c)
    x2 = jnp.sum(flat ** 2, axis=1, keepdims=True)
    e2 = jnp.sum(emb ** 2, axis=1)[None, :]
    return flat.astype(jnp.bfloat16), emb.T.astype(jnp.bfloat16), x2, e2


def kernel(inputs_0, inputs_1, emb0, emb1):
    b0, c0, h0, w0 = inputs_0.shape
    b1, c1, h1, w1 = inputs_1.shape

    idx0, lsum0 = _argmin_level(*_vq_level_setup(inputs_0, emb0), 1024, True)
    idx1, lsum1 = _argmin_level(*_vq_level_setup(inputs_1, emb1), 1024, False)

    q0 = _sc_gather(emb0, idx0).reshape(b0, h0, w0, c0).transpose(0, 3, 1, 2)
    q1 = _sc_gather(emb1, idx1).reshape(b1, h1, w1, c1).transpose(0, 3, 1, 2)

    n0 = b0 * h0 * w0
    n1 = b1 * h1 * w1
    total_loss = _COMMITMENT_COST * (lsum0 / (n0 * c0) + lsum1 / (n1 * c1))
    return total_loss, q0, q1
